# baseline (speedup 1.0000x reference)
import functools

import jax
import jax.numpy as jnp
from jax import lax
from jax.experimental import pallas as pl
from jax.experimental.pallas import tpu as pltpu
from jax.experimental.pallas import tpu_sc as plsc

_SIZES = (1024, 512, 256, 128)
_C = 16
_B = 4
_HW = 512
_P = _B * _HW * _HW
_NW = 32
_PPW = _P // _NW
_CHUNK = 128
_NCHUNK = _PPW // _CHUNK
_NPAIR = 8
_HImask = jnp.uint32(0xFFFF0000)


def _sc_sample(u, v, t0, t1, t2, t3):
    mesh = plsc.VectorSubcoreMesh(core_axis_name="c", subcore_axis_name="s")

    @functools.partial(
        pl.kernel,
        mesh=mesh,
        out_type=jax.ShapeDtypeStruct((_P, _C), jnp.float32),
        compiler_params=pltpu.CompilerParams(use_tc_tiling_on_sc=False),
        scratch_types=[
            pltpu.VMEM((_CHUNK,), jnp.float32),
            pltpu.VMEM((_CHUNK,), jnp.float32),
            pltpu.VMEM((_NPAIR, _CHUNK), jnp.int32),
            pltpu.VMEM((_NPAIR, _CHUNK), jnp.int32),
            pltpu.VMEM((16, _CHUNK), jnp.float32),
            pltpu.VMEM((16, _CHUNK), jnp.float32),
            pltpu.VMEM((_NPAIR * _CHUNK, _C), jnp.uint32),
            pltpu.VMEM((_NPAIR * _CHUNK, _C), jnp.uint32),
            pltpu.VMEM((_CHUNK, _C), jnp.float32),
            pltpu.SemaphoreType.DMA,
            pltpu.SemaphoreType.DMA,
        ],
    )
    def body(u_hbm, v_hbm, t0_hbm, t1_hbm, t2_hbm, t3_hbm, out_hbm,
             u_v, v_v, idxA, idxB, wA, wB, rowsA, rowsB, o_v, semA, semB):
        tabs = (t0_hbm, t1_hbm, t2_hbm, t3_hbm)
        wid = lax.axis_index("s") * 2 + lax.axis_index("c")
        wbase = wid * _PPW

        def fire(g, idx_v, w_v, rows_v, sem):
            base = wbase + g * _CHUNK
            pltpu.sync_copy(u_hbm.at[pl.ds(base, _CHUNK)], u_v)
            pltpu.sync_copy(v_hbm.at[pl.ds(base, _CHUNK)], v_v)

            def grp_body(gi, c2):
                sl = pl.ds(gi * 16, 16)
                uu = u_v[sl]
                vv = v_v[sl]
                for li, s in enumerate(_SIZES):
                    ix = ((2.0 * uu - 1.0 + 1.0) * s - 1.0) * 0.5
                    iy = ((2.0 * vv - 1.0 + 1.0) * s - 1.0) * 0.5
                    x0i = (ix + 1.0).astype(jnp.int32)
                    y0i = (iy + 1.0).astype(jnp.int32)
                    fx = ix - (x0i.astype(jnp.float32) - 1.0)
                    fy = iy - (y0i.astype(jnp.float32) - 1.0)
                    xb = jnp.minimum(jnp.maximum(x0i - 1, 0), s - 1)
                    yc0 = jnp.minimum(jnp.maximum(y0i - 1, 0), s - 1)
                    yc1 = jnp.minimum(jnp.maximum(y0i, 0), s - 1)
                    w0x = jnp.where(x0i >= 1, 1.0 - fx, 0.0)
                    w1x = jnp.where(x0i <= s - 1, fx, 0.0)
                    w0y = jnp.where(y0i >= 1, 1.0 - fy, 0.0)
                    w1y = jnp.where(y0i <= s - 1, fy, 0.0)
                    left = x0i == 0
                    wf0 = jnp.where(left, w1x, w0x)
                    wf1 = jnp.where(left, 0.0, w1x)
                    idx_v[li * 2 + 0, sl] = yc0 * s + xb
                    idx_v[li * 2 + 1, sl] = yc1 * s + xb
                    w_v[li * 4 + 0, sl] = wf0 * w0y
                    w_v[li * 4 + 1, sl] = wf1 * w0y
                    w_v[li * 4 + 2, sl] = wf0 * w1y
                    w_v[li * 4 + 3, sl] = wf1 * w1y
                return c2

            lax.fori_loop(0, _CHUNK // 16, grp_body, 0)
            for li in range(4):
                for yi in range(2):
                    k = li * 2 + yi
                    pltpu.async_copy(
                        tabs[li].at[idx_v.at[k]],
                        rows_v.at[pl.ds(k * _CHUNK, _CHUNK)], sem)

        def process(g, w_v, rows_v, sem):
            pltpu.make_async_copy(
                tabs[0].at[pl.ds(0, _NPAIR * _CHUNK)], rows_v, sem).wait()

            def wgrp_body(gi, c3):
                sl = pl.ds(gi * 16, 16)
                wk = [w_v[k, sl] for k in range(16)]
                for j in range(16):
                    p = gi * 16 + j
                    acc = None
                    for li in range(4):
                        for yi in range(2):
                            r = rows_v[(li * 2 + yi) * _CHUNK + p]
                            f0 = lax.bitcast_convert_type(r << 16, jnp.float32)
                            f1 = lax.bitcast_convert_type(r & _HImask,
                                                          jnp.float32)
                            t = wk[li * 4 + yi * 2][j] * f0 \
                                + wk[li * 4 + yi * 2 + 1][j] * f1
                            acc = t if acc is None else acc + t
                    o_v[p] = acc
                return c3

            lax.fori_loop(0, _CHUNK // 16, wgrp_body, 0)
            base = wbase + g * _CHUNK
            pltpu.sync_copy(o_v, out_hbm.at[pl.ds(base, _CHUNK)])

        fire(0, idxA, wA, rowsA, semA)

        def pair_body(i, carry):
            g0 = 2 * i
            fire(g0 + 1, idxB, wB, rowsB, semB)
            process(g0, wA, rowsA, semA)

            @pl.when(i < _NCHUNK // 2 - 1)
            def _():
                fire(g0 + 2, idxA, wA, rowsA, semA)

            process(g0 + 1, wB, rowsB, semB)
            return carry

        lax.fori_loop(0, _NCHUNK // 2, pair_body, 0)

    return body(u, v, t0, t1, t2, t3)


def _prep_table(m, s):
    mm = m.reshape(_C, s, s)
    b0 = lax.bitcast_convert_type(mm.astype(jnp.bfloat16),
                                  jnp.uint16).astype(jnp.uint32)
    b1 = jnp.roll(b0, -1, axis=2)
    packed = (b0 | (b1 << 16)).reshape(_C, s * s)
    return jnp.transpose(packed)


def kernel(input, mipmap_0, mipmap_1, mipmap_2, mipmap_3):
    u = input[..., 0].reshape(_P)
    v = input[..., 1].reshape(_P)
    tables = [_prep_table(m, s)
              for m, s in zip((mipmap_0, mipmap_1, mipmap_2, mipmap_3), _SIZES)]
    out = _sc_sample(u, v, *tables)
    return out.reshape(_B, _HW, _HW, _C).transpose(0, 3, 1, 2)


# speedup vs baseline: 1.0013x; 1.0013x over previous
import functools

import jax
import jax.numpy as jnp
from jax import lax
from jax.experimental import pallas as pl
from jax.experimental.pallas import tpu as pltpu
from jax.experimental.pallas import tpu_sc as plsc

_SIZES = (1024, 512, 256, 128)
_C = 16
_B = 4
_HW = 512
_P = _B * _HW * _HW
_NW = 32
_PPW = _P // _NW
_CHUNK = 128
_NCHUNK = _PPW // _CHUNK
_NPAIR = 8
_HImask = jnp.uint32(0xFFFF0000)


def _sc_sample(u, v, t0, t1, t2, t3):
    mesh = plsc.VectorSubcoreMesh(core_axis_name="c", subcore_axis_name="s")

    @functools.partial(
        pl.kernel,
        mesh=mesh,
        out_type=jax.ShapeDtypeStruct((_P, _C), jnp.float32),
        compiler_params=pltpu.CompilerParams(use_tc_tiling_on_sc=False),
        scratch_types=[
            pltpu.VMEM((_CHUNK,), jnp.float32),
            pltpu.VMEM((_CHUNK,), jnp.float32),
            pltpu.VMEM((_NPAIR, _CHUNK), jnp.int32),
            pltpu.VMEM((_NPAIR, _CHUNK), jnp.int32),
            pltpu.VMEM((16, _CHUNK), jnp.float32),
            pltpu.VMEM((16, _CHUNK), jnp.float32),
            pltpu.VMEM((_NPAIR * _CHUNK, _C), jnp.uint32),
            pltpu.VMEM((_NPAIR * _CHUNK, _C), jnp.uint32),
            pltpu.VMEM((_CHUNK, _C), jnp.float32),
            pltpu.SemaphoreType.DMA,
            pltpu.SemaphoreType.DMA,
        ],
    )
    def body(u_hbm, v_hbm, t0_hbm, t1_hbm, t2_hbm, t3_hbm, out_hbm,
             u_v, v_v, idxA, idxB, wA, wB, rowsA, rowsB, o_v, semA, semB):
        tabs = (t0_hbm, t1_hbm, t2_hbm, t3_hbm)
        wid = lax.axis_index("s") * 2 + lax.axis_index("c")
        wbase = wid * _PPW

        def fire(g, idx_v, w_v, rows_v, sem):
            base = wbase + g * _CHUNK
            pltpu.sync_copy(u_hbm.at[pl.ds(base, _CHUNK)], u_v)
            pltpu.sync_copy(v_hbm.at[pl.ds(base, _CHUNK)], v_v)

            def grp_body(gi, c2):
                sl = pl.ds(gi * 16, 16)
                uu = u_v[sl]
                vv = v_v[sl]
                for li, s in enumerate(_SIZES):
                    ix = ((2.0 * uu - 1.0 + 1.0) * s - 1.0) * 0.5
                    iy = ((2.0 * vv - 1.0 + 1.0) * s - 1.0) * 0.5
                    x0i = (ix + 1.0).astype(jnp.int32)
                    y0i = (iy + 1.0).astype(jnp.int32)
                    fx = ix - (x0i.astype(jnp.float32) - 1.0)
                    fy = iy - (y0i.astype(jnp.float32) - 1.0)
                    xb = jnp.minimum(jnp.maximum(x0i - 1, 0), s - 1)
                    yc0 = jnp.minimum(jnp.maximum(y0i - 1, 0), s - 1)
                    yc1 = jnp.minimum(jnp.maximum(y0i, 0), s - 1)
                    w0x = jnp.where(x0i >= 1, 1.0 - fx, 0.0)
                    w1x = jnp.where(x0i <= s - 1, fx, 0.0)
                    w0y = jnp.where(y0i >= 1, 1.0 - fy, 0.0)
                    w1y = jnp.where(y0i <= s - 1, fy, 0.0)
                    left = x0i == 0
                    wf0 = jnp.where(left, w1x, w0x)
                    wf1 = jnp.where(left, 0.0, w1x)
                    idx_v[li * 2 + 0, sl] = yc0 * s + xb
                    idx_v[li * 2 + 1, sl] = yc1 * s + xb
                    w_v[li * 4 + 0, sl] = wf0 * w0y
                    w_v[li * 4 + 1, sl] = wf1 * w0y
                    w_v[li * 4 + 2, sl] = wf0 * w1y
                    w_v[li * 4 + 3, sl] = wf1 * w1y
                return c2

            lax.fori_loop(0, _CHUNK // 16, grp_body, 0)
            for li in range(4):
                for yi in range(2):
                    k = li * 2 + yi
                    pltpu.async_copy(
                        tabs[li].at[idx_v.at[k]],
                        rows_v.at[pl.ds(k * _CHUNK, _CHUNK)], sem)

        def process(g, w_v, rows_v, sem):
            pltpu.make_async_copy(
                tabs[0].at[pl.ds(0, _NPAIR * _CHUNK)], rows_v, sem).wait()

            def wgrp_body(gi, c3):
                sl = pl.ds(gi * 16, 16)
                wk = [w_v[k, sl] for k in range(16)]
                for j in range(16):
                    p = gi * 16 + j
                    acc = None
                    for li in range(4):
                        for yi in range(2):
                            r = rows_v[(li * 2 + yi) * _CHUNK + p]
                            f0 = lax.bitcast_convert_type(r << 16, jnp.float32)
                            f1 = lax.bitcast_convert_type(r & _HImask,
                                                          jnp.float32)
                            t = wk[li * 4 + yi * 2][j] * f0 \
                                + wk[li * 4 + yi * 2 + 1][j] * f1
                            acc = t if acc is None else acc + t
                    o_v[p] = acc
                return c3

            lax.fori_loop(0, _CHUNK // 16, wgrp_body, 0)
            base = wbase + g * _CHUNK
            pltpu.sync_copy(o_v, out_hbm.at[pl.ds(base, _CHUNK)])

        fire(0, idxA, wA, rowsA, semA)

        def pair_body(i, carry):
            g0 = 2 * i
            fire(g0 + 1, idxB, wB, rowsB, semB)
            process(g0, wA, rowsA, semA)

            @pl.when(i < _NCHUNK // 2 - 1)
            def _():
                fire(g0 + 2, idxA, wA, rowsA, semA)

            process(g0 + 1, wB, rowsB, semB)
            return carry

        lax.fori_loop(0, _NCHUNK // 2, pair_body, 0)

    return body(u, v, t0, t1, t2, t3)


def _prep_table(m, s):
    mm = m.reshape(_C, s, s)
    u32 = lax.bitcast_convert_type(mm, jnp.uint32)
    b0 = (u32 + 0x7FFF + ((u32 >> 16) & 1)) >> 16
    b1 = jnp.roll(b0, -1, axis=2)
    packed = (b0 | (b1 << 16)).reshape(_C, s * s)
    return jnp.transpose(packed)


def kernel(input, mipmap_0, mipmap_1, mipmap_2, mipmap_3):
    u = input[..., 0].reshape(_P)
    v = input[..., 1].reshape(_P)
    tables = [_prep_table(m, s)
              for m, s in zip((mipmap_0, mipmap_1, mipmap_2, mipmap_3), _SIZES)]
    out = _sc_sample(u, v, *tables)
    return out.reshape(_B, _HW, _HW, _C).transpose(0, 3, 1, 2)


# speedup vs baseline: 1.0192x; 1.0178x over previous
import functools

import jax
import jax.numpy as jnp
from jax import lax
from jax.experimental import pallas as pl
from jax.experimental.pallas import tpu as pltpu
from jax.experimental.pallas import tpu_sc as plsc

_SIZES = (1024, 512, 256, 128)
_C = 16
_B = 4
_HW = 512
_P = _B * _HW * _HW
_NW = 32
_PPW = _P // _NW
_CHUNK = 128
_NCHUNK = _PPW // _CHUNK
_NPAIR = 8
_HImask = jnp.uint32(0xFFFF0000)


def _sc_sample(u, v, t0, t1, t2, t3):
    mesh = plsc.VectorSubcoreMesh(core_axis_name="c", subcore_axis_name="s")

    @functools.partial(
        pl.kernel,
        mesh=mesh,
        out_type=jax.ShapeDtypeStruct((_P, _C), jnp.float32),
        compiler_params=pltpu.CompilerParams(use_tc_tiling_on_sc=False),
        scratch_types=[
            pltpu.VMEM((_CHUNK,), jnp.float32),
            pltpu.VMEM((_CHUNK,), jnp.float32),
            pltpu.VMEM((_NPAIR, _CHUNK), jnp.int32),
            pltpu.VMEM((_NPAIR, _CHUNK), jnp.int32),
            pltpu.VMEM((16, _CHUNK), jnp.float32),
            pltpu.VMEM((16, _CHUNK), jnp.float32),
            pltpu.VMEM((_NPAIR * _CHUNK, _C), jnp.uint32),
            pltpu.VMEM((_NPAIR * _CHUNK, _C), jnp.uint32),
            pltpu.VMEM((_CHUNK, _C), jnp.float32),
            pltpu.SemaphoreType.DMA,
            pltpu.SemaphoreType.DMA,
        ],
    )
    def body(u_hbm, v_hbm, t0_hbm, t1_hbm, t2_hbm, t3_hbm, out_hbm,
             u_v, v_v, idxA, idxB, wA, wB, rowsA, rowsB, o_v, semA, semB):
        tabs = (t0_hbm, t1_hbm, t2_hbm, t3_hbm)
        wid = lax.axis_index("s") * 2 + lax.axis_index("c")
        wbase = wid * _PPW

        def fire(g, idx_v, w_v, rows_v, sem):
            base = wbase + g * _CHUNK
            pltpu.sync_copy(u_hbm.at[pl.ds(base, _CHUNK)], u_v)
            pltpu.sync_copy(v_hbm.at[pl.ds(base, _CHUNK)], v_v)

            def grp_body(gi, c2):
                sl = pl.ds(gi * 16, 16)
                uu = u_v[sl]
                vv = v_v[sl]
                for li, s in enumerate(_SIZES):
                    ix = ((2.0 * uu - 1.0 + 1.0) * s - 1.0) * 0.5
                    iy = ((2.0 * vv - 1.0 + 1.0) * s - 1.0) * 0.5
                    x0i = (ix + 1.0).astype(jnp.int32)
                    y0i = (iy + 1.0).astype(jnp.int32)
                    fx = ix - (x0i.astype(jnp.float32) - 1.0)
                    fy = iy - (y0i.astype(jnp.float32) - 1.0)
                    xb = jnp.minimum(jnp.maximum(x0i - 1, 0), s - 1)
                    yc0 = jnp.minimum(jnp.maximum(y0i - 1, 0), s - 1)
                    yc1 = jnp.minimum(jnp.maximum(y0i, 0), s - 1)
                    w0x = jnp.where(x0i >= 1, 1.0 - fx, 0.0)
                    w1x = jnp.where(x0i <= s - 1, fx, 0.0)
                    w0y = jnp.where(y0i >= 1, 1.0 - fy, 0.0)
                    w1y = jnp.where(y0i <= s - 1, fy, 0.0)
                    left = x0i == 0
                    wf0 = jnp.where(left, w1x, w0x)
                    wf1 = jnp.where(left, 0.0, w1x)
                    idx_v[li * 2 + 0, sl] = yc0 * s + xb
                    idx_v[li * 2 + 1, sl] = yc1 * s + xb
                    w_v[li * 4 + 0, sl] = wf0 * w0y
                    w_v[li * 4 + 1, sl] = wf1 * w0y
                    w_v[li * 4 + 2, sl] = wf0 * w1y
                    w_v[li * 4 + 3, sl] = wf1 * w1y
                return c2

            lax.fori_loop(0, _CHUNK // 16, grp_body, 0)
            for li in range(4):
                for yi in range(2):
                    k = li * 2 + yi
                    pltpu.async_copy(
                        tabs[li].at[idx_v.at[k]],
                        rows_v.at[pl.ds(k * _CHUNK, _CHUNK)], sem)

        def process(g, w_v, rows_v, sem):
            pltpu.make_async_copy(
                tabs[0].at[pl.ds(0, _NPAIR * _CHUNK)], rows_v, sem).wait()

            def wgrp_body(gi, c3):
                sl = pl.ds(gi * 16, 16)
                wk = [w_v[k, sl] for k in range(16)]
                for j in range(16):
                    p = gi * 16 + j
                    acc = None
                    for li in range(4):
                        for yi in range(2):
                            r = rows_v[(li * 2 + yi) * _CHUNK + p]
                            f0 = lax.bitcast_convert_type(r << 16, jnp.float32)
                            f1 = lax.bitcast_convert_type(r & _HImask,
                                                          jnp.float32)
                            t = wk[li * 4 + yi * 2][j] * f0 \
                                + wk[li * 4 + yi * 2 + 1][j] * f1
                            acc = t if acc is None else acc + t
                    o_v[p] = acc
                return c3

            lax.fori_loop(0, _CHUNK // 16, wgrp_body, 0)
            base = wbase + g * _CHUNK
            pltpu.sync_copy(o_v, out_hbm.at[pl.ds(base, _CHUNK)])

        fire(0, idxA, wA, rowsA, semA)

        def pair_body(i, carry):
            g0 = 2 * i
            fire(g0 + 1, idxB, wB, rowsB, semB)
            process(g0, wA, rowsA, semA)

            @pl.when(i < _NCHUNK // 2 - 1)
            def _():
                fire(g0 + 2, idxA, wA, rowsA, semA)

            process(g0 + 1, wB, rowsB, semB)
            return carry

        lax.fori_loop(0, _NCHUNK // 2, pair_body, 0)

    return body(u, v, t0, t1, t2, t3)


def _prep_table(m, s):
    t_lin = jnp.transpose(m.reshape(_C, s * s))
    u32 = lax.bitcast_convert_type(t_lin, jnp.uint32)
    b0 = (u32 + 0x7FFF + ((u32 >> 16) & 1)) >> 16
    b1 = jnp.roll(b0, -1, axis=0)
    return b0 | (b1 << 16)


def kernel(input, mipmap_0, mipmap_1, mipmap_2, mipmap_3):
    u = input[..., 0].reshape(_P)
    v = input[..., 1].reshape(_P)
    tables = [_prep_table(m, s)
              for m, s in zip((mipmap_0, mipmap_1, mipmap_2, mipmap_3), _SIZES)]
    out = _sc_sample(u, v, *tables)
    return out.reshape(_B, _HW, _HW, _C).transpose(0, 3, 1, 2)


# speedup vs baseline: 1.5117x; 1.4832x over previous
import functools

import jax
import jax.numpy as jnp
from jax import lax
from jax.experimental import pallas as pl
from jax.experimental.pallas import tpu as pltpu
from jax.experimental.pallas import tpu_sc as plsc

_SIZES = (1024, 512, 256, 128)
_C = 16
_B = 4
_HW = 512
_P = _B * _HW * _HW
_NW = 32
_PPW = _P // _NW
_CHUNK = 128
_NCHUNK = _PPW // _CHUNK
_NG = 16


def _sc_sample(uv, t0, t1, t2, t3):
    mesh = plsc.VectorSubcoreMesh(core_axis_name="c", subcore_axis_name="s")

    @functools.partial(
        pl.kernel,
        mesh=mesh,
        out_type=jax.ShapeDtypeStruct((_P, _C), jnp.float32),
        compiler_params=pltpu.CompilerParams(use_tc_tiling_on_sc=False),
        scratch_types=[
            pltpu.VMEM((2, _CHUNK), jnp.float32),
            pltpu.VMEM((_NG, _CHUNK), jnp.int32),
            pltpu.VMEM((_NG, _CHUNK), jnp.int32),
            pltpu.VMEM((_NG, _CHUNK), jnp.float32),
            pltpu.VMEM((_NG, _CHUNK), jnp.float32),
            pltpu.VMEM((_NG * _CHUNK, _C), jnp.float32),
            pltpu.VMEM((_NG * _CHUNK, _C), jnp.float32),
            pltpu.VMEM((_CHUNK, _C), jnp.float32),
            pltpu.VMEM((_CHUNK, _C), jnp.float32),
            pltpu.SemaphoreType.DMA,
            pltpu.SemaphoreType.DMA,
            pltpu.SemaphoreType.DMA,
        ],
    )
    def body(uv_hbm, t0_hbm, t1_hbm, t2_hbm, t3_hbm, out_hbm,
             uv_v, idxA, idxB, wA, wB, rowsA, rowsB, oA, oB, semA, semB, semO):
        tabs = (t0_hbm, t1_hbm, t2_hbm, t3_hbm)
        wid = lax.axis_index("s") * 2 + lax.axis_index("c")
        wbase = wid * _PPW

        def fire(g, idx_v, w_v, rows_v, sem):
            base = wbase + g * _CHUNK
            pltpu.sync_copy(uv_hbm.at[:, pl.ds(base, _CHUNK)], uv_v)

            def grp_body(gi, c2):
                sl = pl.ds(gi * 16, 16)
                uu = uv_v[0, sl]
                vv = uv_v[1, sl]
                for li, s in enumerate(_SIZES):
                    ix = ((2.0 * uu - 1.0 + 1.0) * s - 1.0) * 0.5
                    iy = ((2.0 * vv - 1.0 + 1.0) * s - 1.0) * 0.5
                    x0i = (ix + 1.0).astype(jnp.int32)
                    y0i = (iy + 1.0).astype(jnp.int32)
                    fx = ix - (x0i.astype(jnp.float32) - 1.0)
                    fy = iy - (y0i.astype(jnp.float32) - 1.0)
                    xc0 = jnp.maximum(x0i - 1, 0)
                    xc1 = jnp.minimum(jnp.maximum(x0i, 0), s - 1)
                    yc0 = jnp.maximum(y0i - 1, 0)
                    yc1 = jnp.minimum(jnp.maximum(y0i, 0), s - 1)
                    w0x = jnp.where(x0i >= 1, 1.0 - fx, 0.0)
                    w1x = jnp.where(x0i <= s - 1, fx, 0.0)
                    w0y = jnp.where(y0i >= 1, 1.0 - fy, 0.0)
                    w1y = jnp.where(y0i <= s - 1, fy, 0.0)
                    r0 = yc0 * s
                    r1 = yc1 * s
                    idx_v[li * 4 + 0, sl] = r0 + xc0
                    idx_v[li * 4 + 1, sl] = r0 + xc1
                    idx_v[li * 4 + 2, sl] = r1 + xc0
                    idx_v[li * 4 + 3, sl] = r1 + xc1
                    w_v[li * 4 + 0, sl] = w0x * w0y
                    w_v[li * 4 + 1, sl] = w1x * w0y
                    w_v[li * 4 + 2, sl] = w0x * w1y
                    w_v[li * 4 + 3, sl] = w1x * w1y
                return c2

            lax.fori_loop(0, _CHUNK // 16, grp_body, 0)
            for li in range(4):
                for c in range(4):
                    k = li * 4 + c
                    pltpu.async_copy(
                        tabs[li].at[idx_v.at[k]],
                        rows_v.at[pl.ds(k * _CHUNK, _CHUNK)], sem)

        def process(g, w_v, rows_v, sem, o_v):
            pltpu.make_async_copy(
                out_hbm.at[pl.ds(0, _NG * _CHUNK)], rows_v, sem).wait()

            @pl.when(g >= 2)
            def _():
                pltpu.make_async_copy(
                    o_v, out_hbm.at[pl.ds(0, _CHUNK)], semO).wait()

            def wgrp_body(gi, c3):
                sl = pl.ds(gi * 16, 16)
                wk = [w_v[k, sl] for k in range(_NG)]
                for j in range(16):
                    p = gi * 16 + j
                    acc = wk[0][j] * rows_v[p]
                    for k in range(1, _NG):
                        acc = acc + wk[k][j] * rows_v[k * _CHUNK + p]
                    o_v[p] = acc
                return c3

            lax.fori_loop(0, _CHUNK // 16, wgrp_body, 0)
            base = wbase + g * _CHUNK
            pltpu.async_copy(o_v, out_hbm.at[pl.ds(base, _CHUNK)], semO)

        fire(0, idxA, wA, rowsA, semA)

        def pair_body(i, carry):
            g0 = 2 * i
            fire(g0 + 1, idxB, wB, rowsB, semB)
            process(g0, wA, rowsA, semA, oA)

            @pl.when(i < _NCHUNK // 2 - 1)
            def _():
                fire(g0 + 2, idxA, wA, rowsA, semA)

            process(g0 + 1, wB, rowsB, semB, oB)
            return carry

        lax.fori_loop(0, _NCHUNK // 2, pair_body, 0)
        pltpu.make_async_copy(oA, out_hbm.at[pl.ds(0, _CHUNK)], semO).wait()
        pltpu.make_async_copy(oB, out_hbm.at[pl.ds(0, _CHUNK)], semO).wait()

    return body(uv, t0, t1, t2, t3)


def _prep_table(m, s):
    return jnp.transpose(m.reshape(_C, s * s))


def kernel(input, mipmap_0, mipmap_1, mipmap_2, mipmap_3):
    uv = jnp.stack([input[..., 0].reshape(_P), input[..., 1].reshape(_P)])
    tables = [_prep_table(m, s)
              for m, s in zip((mipmap_0, mipmap_1, mipmap_2, mipmap_3), _SIZES)]
    out = _sc_sample(uv, *tables)
    return out.reshape(_B, _HW, _HW, _C).transpose(0, 3, 1, 2)
